# baseline (speedup 1.0000x reference)
import jax
import jax.numpy as jnp
from jax import lax
from jax.experimental import pallas as pl
from jax.experimental.pallas import tpu as pltpu
from jax.experimental.pallas import tpu_sc as plsc

EMBED = 64
NC, NS = 2, 16
NW = NC * NS
IDXW = 128
CHUNK = 1024
GROUPS = CHUNK // IDXW


def _sc_gather(table, idx2d):
    num_indices = idx2d.shape[0] * idx2d.shape[1]
    rows_per_w = num_indices // NW
    grps_per_w = rows_per_w // IDXW
    chunks_per_w = rows_per_w // CHUNK
    mesh = plsc.VectorSubcoreMesh(core_axis_name="c", subcore_axis_name="s")

    @pl.kernel(
        out_type=jax.ShapeDtypeStruct((num_indices, EMBED), table.dtype),
        mesh=mesh,
        scratch_types=[
            pltpu.VMEM((GROUPS, IDXW), jnp.int32),
            pltpu.VMEM((CHUNK, EMBED), jnp.float32),
            pltpu.SemaphoreType.DMA,
        ],
        compiler_params=pltpu.CompilerParams(use_tc_tiling_on_sc=False,
                                             skip_device_barrier=True),
    )
    def emb_gather(table_hbm, idx_hbm, out_hbm, idx_v, rows_v, sem):
        wid = lax.axis_index("s") * NC + lax.axis_index("c")

        @pl.loop(0, chunks_per_w)
        def _(c):
            grp0 = pl.multiple_of(wid * grps_per_w + c * GROUPS, GROUPS)
            pltpu.sync_copy(idx_hbm.at[pl.ds(grp0, GROUPS)], idx_v)
            for g in range(GROUPS):
                pltpu.async_copy(
                    table_hbm.at[idx_v.at[g]],
                    rows_v.at[pl.ds(g * IDXW, IDXW)],
                    sem,
                )
            pltpu.make_async_copy(
                table_hbm.at[pl.ds(0, CHUNK)], rows_v, sem
            ).wait()
            row0 = pl.multiple_of(wid * rows_per_w + c * CHUNK, CHUNK)
            pltpu.sync_copy(rows_v, out_hbm.at[pl.ds(row0, CHUNK)])

    return emb_gather(table, idx2d)


@jax.jit
def kernel(sequence, table):
    b, s = sequence.shape
    idx2d = sequence.reshape(b * s // IDXW, IDXW).astype(jnp.int32)
    out = _sc_gather(table, idx2d)
    return out.reshape(b, s, EMBED)


# speedup vs baseline: 1.0010x; 1.0010x over previous
import jax
import jax.numpy as jnp
from jax import lax
from jax.experimental import pallas as pl
from jax.experimental.pallas import tpu as pltpu
from jax.experimental.pallas import tpu_sc as plsc

EMBED = 64
NC, NS = 2, 16
NW = NC * NS
IDXW = 128
CHUNK = 1024
GROUPS = CHUNK // IDXW


def _sc_gather(table, idx2d):
    num_indices = idx2d.shape[0] * idx2d.shape[1]
    rows_per_w = num_indices // NW
    grps_per_w = rows_per_w // IDXW
    chunks_per_w = rows_per_w // CHUNK
    mesh = plsc.VectorSubcoreMesh(core_axis_name="c", subcore_axis_name="s")

    @pl.kernel(
        out_type=jax.ShapeDtypeStruct((num_indices, EMBED), table.dtype),
        mesh=mesh,
        scratch_types=[
            pltpu.VMEM((GROUPS, IDXW), jnp.float32),
            pltpu.VMEM((GROUPS, IDXW), jnp.int32),
            pltpu.VMEM((CHUNK, EMBED), jnp.float32),
            pltpu.SemaphoreType.DMA,
        ],
        compiler_params=pltpu.CompilerParams(use_tc_tiling_on_sc=False),
    )
    def emb_gather(table_hbm, idx_hbm, out_hbm, idx_vf, idx_v, rows_v, sem):
        wid = lax.axis_index("s") * NC + lax.axis_index("c")

        @pl.loop(0, chunks_per_w)
        def _(c):
            grp0 = pl.multiple_of(wid * grps_per_w + c * GROUPS, GROUPS)
            pltpu.sync_copy(idx_hbm.at[pl.ds(grp0, GROUPS)], idx_vf)
            for g in range(GROUPS):
                for j in range(IDXW // 16):
                    sl = pl.ds(j * 16, 16)
                    idx_v[g, sl] = idx_vf[g, sl].astype(jnp.int32)
            for g in range(GROUPS):
                pltpu.async_copy(
                    table_hbm.at[idx_v.at[g]],
                    rows_v.at[pl.ds(g * IDXW, IDXW)],
                    sem,
                )
            pltpu.make_async_copy(
                table_hbm.at[pl.ds(0, CHUNK)], rows_v, sem
            ).wait()
            row0 = pl.multiple_of(wid * rows_per_w + c * CHUNK, CHUNK)
            pltpu.sync_copy(rows_v, out_hbm.at[pl.ds(row0, CHUNK)])

    return emb_gather(table, idx2d)


@jax.jit
def kernel(sequence, table):
    b, s = sequence.shape
    idx2d = sequence.reshape(b * s // IDXW, IDXW).astype(jnp.float32)
    out = _sc_gather(table, idx2d)
    return out.reshape(b, s, EMBED)


# speedup vs baseline: 1.0020x; 1.0010x over previous
import jax
import jax.numpy as jnp
from jax import lax
from jax.experimental import pallas as pl
from jax.experimental.pallas import tpu as pltpu
from jax.experimental.pallas import tpu_sc as plsc

EMBED = 64
NC, NS = 2, 16
NW = NC * NS
IDXW = 128
CHUNK = 1024
GROUPS = CHUNK // IDXW


def _sc_gather(table, idx2d):
    num_indices = idx2d.shape[0] * idx2d.shape[1]
    rows_per_w = num_indices // NW
    grps_per_w = rows_per_w // IDXW
    chunks_per_w = rows_per_w // CHUNK
    mesh = plsc.VectorSubcoreMesh(core_axis_name="c", subcore_axis_name="s")

    @pl.kernel(
        out_type=jax.ShapeDtypeStruct((num_indices, EMBED), table.dtype),
        mesh=mesh,
        scratch_types=[
            pltpu.VMEM((GROUPS, IDXW), jnp.int32),
            pltpu.VMEM((CHUNK, EMBED), jnp.float32),
            pltpu.SemaphoreType.DMA,
        ],
        compiler_params=pltpu.CompilerParams(use_tc_tiling_on_sc=False),
    )
    def emb_gather(table_hbm, idx_hbm, out_hbm, idx_v, rows_v, sem):
        wid = lax.axis_index("s") * NC + lax.axis_index("c")

        @pl.loop(0, chunks_per_w)
        def _(c):
            grp0 = pl.multiple_of(wid * grps_per_w + c * GROUPS, GROUPS)
            pltpu.sync_copy(idx_hbm.at[pl.ds(grp0, GROUPS)], idx_v)
            for g in range(GROUPS):
                pltpu.async_copy(
                    table_hbm.at[idx_v.at[g]],
                    rows_v.at[pl.ds(g * IDXW, IDXW)],
                    sem,
                )
            pltpu.make_async_copy(
                table_hbm.at[pl.ds(0, CHUNK)], rows_v, sem
            ).wait()
            row0 = pl.multiple_of(wid * rows_per_w + c * CHUNK, CHUNK)
            pltpu.sync_copy(rows_v, out_hbm.at[pl.ds(row0, CHUNK)])

    return emb_gather(table, idx2d)


@jax.jit
def kernel(sequence, table):
    b, s = sequence.shape
    idx2d = sequence.reshape(b * s // IDXW, IDXW).astype(jnp.int32)
    out = _sc_gather(table, idx2d)
    return out.reshape(b, s, EMBED)
